# Initial kernel scaffold; baseline (speedup 1.0000x reference)
import jax
import jax.numpy as jnp
from jax.experimental import pallas as pl


def kernel(x, edge_index, timestep, batch_map, obj_x, obj_pos, nucleotide_mask, central_mask, backbone_mask, obj_batch, W_node, b_node, W_cond, b_cond, W_time, b_time, W_c1, b_c1, W_c2, b_c2, W_out, b_out, W_edge, b_edge):
    raise NotImplementedError("write your pallas kernel here")


# speedup vs baseline: 28.0140x; 28.0140x over previous
import functools

import numpy as np
import jax
import jax.numpy as jnp
from jax import lax
from jax.experimental import pallas as pl
from jax.experimental.pallas import tpu as pltpu
from jax.experimental.pallas import tpu_sc as plsc

N = 1024
E = 32768
G = 8
ND = 131
NDP = 136
HID = 128
NC = 2
NS = 16
NW = NC * NS
EPW = E // NW
ECH = 128
NCHUNK = EPW // ECH
P = N * (N - 1) // 2
PPW = P // NW
KT = PPW // 16

_mesh = plsc.VectorSubcoreMesh(core_axis_name="c", subcore_axis_name="s")

_tri_r, _tri_c = np.triu_indices(N, k=1)
_TRI_R = np.ascontiguousarray(_tri_r.astype(np.int32).reshape(NW, PPW))
_TRI_C = np.ascontiguousarray(_tri_c.astype(np.int32).reshape(NW, PPW))


@functools.partial(
    pl.kernel,
    out_type=jax.ShapeDtypeStruct((NC, N, HID), jnp.float32),
    mesh=_mesh,
    compiler_params=pltpu.CompilerParams(needs_layout_passes=False),
    scratch_types=[
        pltpu.VMEM((NCHUNK, ECH), jnp.int32),
        pltpu.VMEM((ECH, HID), jnp.float32),
        pltpu.VMEM((64, HID), jnp.float32),
        pltpu.VMEM_SHARED((N, HID), jnp.float32),
    ],
)
def _deg_kernel(d_hbm, out_hbm, didx_v, ones_v, buf_v, tab_sh):
    c = lax.axis_index("c")
    s = lax.axis_index("s")
    wid = c * NS + s
    one16 = jnp.full((16,), 1.0, jnp.float32)
    zero16 = jnp.zeros((16,), jnp.float32)
    for i in range(ECH):
        for j in range(HID // 16):
            ones_v[i, pl.ds(j * 16, 16)] = one16
    for i in range(64):
        for j in range(HID // 16):
            buf_v[i, pl.ds(j * 16, 16)] = zero16
    pltpu.sync_copy(d_hbm.at[wid], didx_v)
    pltpu.sync_copy(buf_v, tab_sh.at[pl.ds(s * 64, 64)])
    plsc.subcore_barrier()
    for j in range(NCHUNK):
        pltpu.sync_copy(ones_v, tab_sh.at[didx_v.at[j]], add=True)
    plsc.subcore_barrier()
    pltpu.sync_copy(tab_sh.at[pl.ds(s * 64, 64)],
                    out_hbm.at[c, pl.ds(s * 64, 64)])


@functools.partial(
    pl.kernel,
    out_type=jax.ShapeDtypeStruct((NC, N, HID), jnp.float32),
    mesh=_mesh,
    compiler_params=pltpu.CompilerParams(needs_layout_passes=False),
    scratch_types=[
        pltpu.VMEM((NCHUNK, ECH), jnp.int32),
        pltpu.VMEM((NCHUNK, ECH), jnp.int32),
        pltpu.VMEM((ECH, HID), jnp.float32),
        pltpu.VMEM((64, HID), jnp.float32),
        pltpu.VMEM_SHARED((N, HID), jnp.float32),
        pltpu.SemaphoreType.DMA,
    ],
)
def _conv_kernel(y_hbm, s_hbm, d_hbm, out_hbm, sidx_v, didx_v, rows_v, zbuf_v,
                 agg_sh, sem):
    c = lax.axis_index("c")
    s = lax.axis_index("s")
    wid = c * NS + s
    zero16 = jnp.zeros((16,), jnp.float32)
    for i in range(64):
        for j in range(HID // 16):
            zbuf_v[i, pl.ds(j * 16, 16)] = zero16
    pltpu.sync_copy(s_hbm.at[wid], sidx_v)
    pltpu.sync_copy(d_hbm.at[wid], didx_v)
    pltpu.sync_copy(zbuf_v, agg_sh.at[pl.ds(s * 64, 64)])
    plsc.subcore_barrier()
    for j in range(NCHUNK):
        pltpu.async_copy(y_hbm.at[sidx_v.at[j]], rows_v, sem).wait()
        pltpu.sync_copy(rows_v, agg_sh.at[didx_v.at[j]], add=True)
    plsc.subcore_barrier()
    pltpu.sync_copy(agg_sh.at[pl.ds(s * 64, 64)],
                    out_hbm.at[c, pl.ds(s * 64, 64)])


@functools.partial(
    pl.kernel,
    out_type=jax.ShapeDtypeStruct((P,), jnp.float32),
    mesh=_mesh,
    compiler_params=pltpu.CompilerParams(needs_layout_passes=False),
    scratch_types=[
        pltpu.VMEM((N,), jnp.float32),
        pltpu.VMEM((N,), jnp.float32),
        pltpu.VMEM((PPW,), jnp.int32),
        pltpu.VMEM((PPW,), jnp.int32),
        pltpu.VMEM((PPW,), jnp.float32),
    ],
)
def _triu_kernel(a_hbm, c_hbm, ri_hbm, ci_hbm, out_hbm, a_v, c_v, r_v, ci_v, o_v):
    c = lax.axis_index("c")
    s = lax.axis_index("s")
    wid = c * NS + s
    pltpu.sync_copy(a_hbm, a_v)
    pltpu.sync_copy(c_hbm, c_v)
    pltpu.sync_copy(ri_hbm.at[wid], r_v)
    pltpu.sync_copy(ci_hbm.at[wid], ci_v)

    def body(k, carry):
        off = k * 16
        r16 = r_v[pl.ds(off, 16)]
        c16 = ci_v[pl.ds(off, 16)]
        av = plsc.load_gather(a_v, [r16])
        cv = plsc.load_gather(c_v, [c16])
        o_v[pl.ds(off, 16)] = av + cv
        return carry

    lax.fori_loop(0, KT, body, 0)
    pltpu.sync_copy(o_v, out_hbm.at[pl.ds(wid * PPW, PPW)])


def _tc_pre_body(x_ref, cond_ref, ts_ref, bm_ref, ob_ref, nuc_ref, cen_ref,
                 bb_ref, deg_ref, Wn_ref, bn_ref, Wc_ref, bc_ref, Wt_ref,
                 bt_ref, W1_ref, b1_ref, y1_ref, dinv_ref):
    f32 = jnp.float32
    t = ts_ref[...].astype(f32)
    j64 = lax.broadcasted_iota(jnp.int32, (1, HID // 2), 1).astype(f32)
    freqs = jnp.exp((-np.log(10000.0) / (HID // 2)) * j64)
    args = t * freqs
    time_emb = jnp.concatenate([jnp.cos(args), jnp.sin(args)], axis=1)
    Wn = Wn_ref[...]
    bn = bn_ref[...]
    node_emb = jnp.dot(x_ref[...], Wn, preferred_element_type=f32) + bn
    cond_emb = jnp.dot(cond_ref[...], Wn, preferred_element_type=f32) + bn
    nuc = nuc_ref[...]
    is_left = nuc == 0
    is_right = nuc == 2
    is_base = bb_ref[...] == 0
    is_central = cen_ref[...] != 0
    is_cond = is_left | (is_central & is_base) | (is_right & is_base)
    m = is_cond.astype(f32)
    oh_obj = (ob_ref[...] == lax.broadcasted_iota(jnp.int32, (G, N), 0))
    oh_obj = oh_obj.astype(f32)
    num = jnp.dot(oh_obj, cond_emb * m, preferred_element_type=f32)
    cnt = jnp.dot(oh_obj, m, preferred_element_type=f32)
    pooled = jnp.where(cnt > 0, num / jnp.maximum(cnt, 1.0), 0.0)
    time_h = jnp.dot(time_emb, Wt_ref[...], preferred_element_type=f32) + bt_ref[...]
    cond_h = jnp.dot(pooled, Wc_ref[...], preferred_element_type=f32) + bc_ref[...]
    both = time_h + cond_h
    oh_bm = (bm_ref[...] == lax.broadcasted_iota(jnp.int32, (N, G), 1)).astype(f32)
    ne = node_emb + jnp.dot(oh_bm, both, preferred_element_type=f32)
    deg = deg_ref[0, :, :1] + deg_ref[1, :, :1] + 1.0
    dinv = lax.rsqrt(jnp.maximum(deg, 1.0))
    z1 = jnp.dot(ne, W1_ref[...], preferred_element_type=f32)
    y1_ref[...] = dinv * z1
    dinv_ref[...] = dinv


def _tc_mid_body(agg_ref, y1_ref, dinv_ref, b1_ref, W2_ref, y2_ref):
    f32 = jnp.float32
    dinv = dinv_ref[...]
    h1 = jnp.maximum(dinv * (agg_ref[0] + agg_ref[1] + y1_ref[...]) + b1_ref[...], 0.0)
    z2 = jnp.dot(h1, W2_ref[...], preferred_element_type=f32)
    y2_ref[...] = dinv * z2


def _tc_out_body(agg_ref, y2_ref, dinv_ref, b2_ref, Wo_ref, bo_ref, We1_ref,
                 We2_ref, be_ref, nnp_ref, a_ref, c_ref):
    f32 = jnp.float32
    dinv = dinv_ref[...]
    h = jnp.maximum(dinv * (agg_ref[0] + agg_ref[1] + y2_ref[...]) + b2_ref[...], 0.0)
    nnp_ref[...] = jnp.dot(h, Wo_ref[...], preferred_element_type=f32) + bo_ref[...]
    a_ref[...] = jnp.dot(h, We1_ref[...], preferred_element_type=f32) + be_ref[...]
    c_ref[...] = jnp.dot(h, We2_ref[...], preferred_element_type=f32)


def kernel(x, edge_index, timestep, batch_map, obj_x, obj_pos, nucleotide_mask,
           central_mask, backbone_mask, obj_batch, W_node, b_node, W_cond,
           b_cond, W_time, b_time, W_c1, b_c1, W_c2, b_c2, W_out, b_out,
           W_edge, b_edge):
    f32 = jnp.float32
    i32 = jnp.int32
    pad5 = jnp.zeros((N, NDP - ND), f32)
    x_pad = jnp.concatenate([x, pad5], axis=1)
    cond_pad = jnp.concatenate([obj_x, obj_pos, pad5], axis=1)
    Wn_pad = jnp.concatenate([W_node, jnp.zeros((NDP - ND, HID), f32)], axis=0)
    Wo_pad = jnp.concatenate([W_out, jnp.zeros((HID, NDP - ND), f32)], axis=1)
    bo_pad = jnp.concatenate([b_out, jnp.zeros((NDP - ND,), f32)]).reshape(1, NDP)

    src3 = edge_index[0].reshape(NW, NCHUNK, ECH)
    dst3 = edge_index[1].reshape(NW, NCHUNK, ECH)

    deg = _deg_kernel(dst3)

    y1, dinv = pl.pallas_call(
        _tc_pre_body,
        out_shape=(jax.ShapeDtypeStruct((N, HID), f32),
                   jax.ShapeDtypeStruct((N, 1), f32)),
    )(x_pad, cond_pad, timestep.reshape(G, 1),
      batch_map.reshape(N, 1), obj_batch.reshape(1, N),
      nucleotide_mask.reshape(N, 1),
      central_mask.astype(i32).reshape(N, 1),
      backbone_mask.astype(i32).reshape(N, 1),
      deg,
      Wn_pad, b_node.reshape(1, HID), W_cond, b_cond.reshape(1, HID),
      W_time, b_time.reshape(1, HID), W_c1, b_c1.reshape(1, HID))

    agg1 = _conv_kernel(y1, src3, dst3)

    y2 = pl.pallas_call(
        _tc_mid_body,
        out_shape=jax.ShapeDtypeStruct((N, HID), f32),
    )(agg1, y1, dinv, b_c1.reshape(1, HID), W_c2)

    agg2 = _conv_kernel(y2, src3, dst3)

    nnp, a_col, c_col = pl.pallas_call(
        _tc_out_body,
        out_shape=(jax.ShapeDtypeStruct((N, NDP), f32),
                   jax.ShapeDtypeStruct((N, 1), f32),
                   jax.ShapeDtypeStruct((N, 1), f32)),
    )(agg2, y2, dinv, b_c2.reshape(1, HID), Wo_pad, bo_pad,
      W_edge[:HID], W_edge[HID:], b_edge.reshape(1, 1))

    edge_logits = _triu_kernel(a_col.reshape(N), c_col.reshape(N),
                               jnp.asarray(_TRI_R), jnp.asarray(_TRI_C))

    return (nnp[:, :ND], edge_logits)


# speedup vs baseline: 30.9112x; 1.1034x over previous
import functools

import numpy as np
import jax
import jax.numpy as jnp
from jax import lax
from jax.experimental import pallas as pl
from jax.experimental.pallas import tpu as pltpu
from jax.experimental.pallas import tpu_sc as plsc

N = 1024
E = 32768
G = 8
ND = 131
NDP = 136
HID = 128
NC = 2
NS = 16
NW = NC * NS
EPW = E // NW
ECH = 128
NCHUNK = EPW // ECH
P = N * (N - 1) // 2
PPW = P // NW
KT = PPW // 16

_mesh = plsc.VectorSubcoreMesh(core_axis_name="c", subcore_axis_name="s")

_tri_r, _tri_c = np.triu_indices(N, k=1)
_TRI_R = np.ascontiguousarray(_tri_r.astype(np.int32).reshape(NW, PPW))
_TRI_C = np.ascontiguousarray(_tri_c.astype(np.int32).reshape(NW, PPW))


@functools.partial(
    pl.kernel,
    out_type=jax.ShapeDtypeStruct((NC, N, HID), jnp.float32),
    mesh=_mesh,
    compiler_params=pltpu.CompilerParams(needs_layout_passes=False),
    scratch_types=[
        pltpu.VMEM((NCHUNK, ECH), jnp.int32),
        pltpu.VMEM((ECH, HID), jnp.float32),
        pltpu.VMEM((64, HID), jnp.float32),
        pltpu.VMEM_SHARED((N, HID), jnp.float32),
    ],
)
def _deg_kernel(d_hbm, out_hbm, didx_v, ones_v, buf_v, tab_sh):
    c = lax.axis_index("c")
    s = lax.axis_index("s")
    wid = c * NS + s
    one16 = jnp.full((16,), 1.0, jnp.float32)
    zero16 = jnp.zeros((16,), jnp.float32)
    for i in range(ECH):
        for j in range(HID // 16):
            ones_v[i, pl.ds(j * 16, 16)] = one16
    for i in range(64):
        for j in range(HID // 16):
            buf_v[i, pl.ds(j * 16, 16)] = zero16
    pltpu.sync_copy(d_hbm.at[wid], didx_v)
    pltpu.sync_copy(buf_v, tab_sh.at[pl.ds(s * 64, 64)])
    plsc.subcore_barrier()
    for j in range(NCHUNK):
        pltpu.sync_copy(ones_v, tab_sh.at[didx_v.at[j]], add=True)
    plsc.subcore_barrier()
    pltpu.sync_copy(tab_sh.at[pl.ds(s * 64, 64)],
                    out_hbm.at[c, pl.ds(s * 64, 64)])


@functools.partial(
    pl.kernel,
    out_type=jax.ShapeDtypeStruct((NC, N, HID), jnp.float32),
    mesh=_mesh,
    compiler_params=pltpu.CompilerParams(needs_layout_passes=False),
    scratch_types=[
        pltpu.VMEM((NCHUNK, ECH), jnp.int32),
        pltpu.VMEM((NCHUNK, ECH), jnp.int32),
        pltpu.VMEM((ECH, HID), jnp.float32),
        pltpu.VMEM((ECH, HID), jnp.float32),
        pltpu.VMEM((64, HID), jnp.float32),
        pltpu.VMEM_SHARED((N, HID), jnp.float32),
        pltpu.SemaphoreType.DMA,
        pltpu.SemaphoreType.DMA,
    ],
)
def _conv_kernel(y_hbm, s_hbm, d_hbm, out_hbm, sidx_v, didx_v, rows_a, rows_b,
                 zbuf_v, agg_sh, sem_a, sem_b):
    c = lax.axis_index("c")
    s = lax.axis_index("s")
    wid = c * NS + s
    zero16 = jnp.zeros((16,), jnp.float32)
    for i in range(64):
        for j in range(HID // 16):
            zbuf_v[i, pl.ds(j * 16, 16)] = zero16
    pltpu.sync_copy(s_hbm.at[wid], sidx_v)
    pltpu.sync_copy(d_hbm.at[wid], didx_v)
    pltpu.sync_copy(zbuf_v, agg_sh.at[pl.ds(s * 64, 64)])
    plsc.subcore_barrier()
    bufs = (rows_a, rows_b)
    sems = (sem_a, sem_b)
    copies = [pltpu.async_copy(y_hbm.at[sidx_v.at[0]], rows_a, sem_a), None]
    for j in range(NCHUNK):
        b = j % 2
        copies[b].wait()
        if j + 1 < NCHUNK:
            nb = (j + 1) % 2
            copies[nb] = pltpu.async_copy(y_hbm.at[sidx_v.at[j + 1]],
                                          bufs[nb], sems[nb])
        pltpu.sync_copy(bufs[b], agg_sh.at[didx_v.at[j]], add=True)
    plsc.subcore_barrier()
    pltpu.sync_copy(agg_sh.at[pl.ds(s * 64, 64)],
                    out_hbm.at[c, pl.ds(s * 64, 64)])


@functools.partial(
    pl.kernel,
    out_type=jax.ShapeDtypeStruct((P,), jnp.float32),
    mesh=_mesh,
    compiler_params=pltpu.CompilerParams(needs_layout_passes=False),
    scratch_types=[
        pltpu.VMEM((N,), jnp.float32),
        pltpu.VMEM((N,), jnp.float32),
        pltpu.VMEM((PPW,), jnp.int32),
        pltpu.VMEM((PPW,), jnp.int32),
        pltpu.VMEM((PPW,), jnp.float32),
    ],
)
def _triu_kernel(a_hbm, c_hbm, ri_hbm, ci_hbm, out_hbm, a_v, c_v, r_v, ci_v, o_v):
    c = lax.axis_index("c")
    s = lax.axis_index("s")
    wid = c * NS + s
    pltpu.sync_copy(a_hbm, a_v)
    pltpu.sync_copy(c_hbm, c_v)
    pltpu.sync_copy(ri_hbm.at[wid], r_v)
    pltpu.sync_copy(ci_hbm.at[wid], ci_v)

    def body(k, carry):
        off = k * 16
        r16 = r_v[pl.ds(off, 16)]
        c16 = ci_v[pl.ds(off, 16)]
        av = plsc.load_gather(a_v, [r16])
        cv = plsc.load_gather(c_v, [c16])
        o_v[pl.ds(off, 16)] = av + cv
        return carry

    lax.fori_loop(0, KT, body, 0)
    pltpu.sync_copy(o_v, out_hbm.at[pl.ds(wid * PPW, PPW)])


def _tc_pre_body(x_ref, cond_ref, ts_ref, bm_ref, ob_ref, nuc_ref, cen_ref,
                 bb_ref, Wn_ref, bn_ref, Wc_ref, bc_ref, Wt_ref,
                 bt_ref, W1_ref, z1_ref):
    f32 = jnp.float32
    t = ts_ref[...].astype(f32)
    j64 = lax.broadcasted_iota(jnp.int32, (1, HID // 2), 1).astype(f32)
    freqs = jnp.exp((-np.log(10000.0) / (HID // 2)) * j64)
    args = t * freqs
    time_emb = jnp.concatenate([jnp.cos(args), jnp.sin(args)], axis=1)
    Wn = Wn_ref[...]
    bn = bn_ref[...]
    node_emb = jnp.dot(x_ref[...], Wn, preferred_element_type=f32) + bn
    cond_emb = jnp.dot(cond_ref[...], Wn, preferred_element_type=f32) + bn
    nuc = nuc_ref[...]
    is_left = nuc == 0
    is_right = nuc == 2
    is_base = bb_ref[...] == 0
    is_central = cen_ref[...] != 0
    is_cond = is_left | (is_central & is_base) | (is_right & is_base)
    m = is_cond.astype(f32)
    oh_obj = (ob_ref[...] == lax.broadcasted_iota(jnp.int32, (G, N), 0))
    oh_obj = oh_obj.astype(f32)
    num = jnp.dot(oh_obj, cond_emb * m, preferred_element_type=f32)
    cnt = jnp.dot(oh_obj, m, preferred_element_type=f32)
    pooled = jnp.where(cnt > 0, num / jnp.maximum(cnt, 1.0), 0.0)
    time_h = jnp.dot(time_emb, Wt_ref[...], preferred_element_type=f32) + bt_ref[...]
    cond_h = jnp.dot(pooled, Wc_ref[...], preferred_element_type=f32) + bc_ref[...]
    both = time_h + cond_h
    oh_bm = (bm_ref[...] == lax.broadcasted_iota(jnp.int32, (N, G), 1)).astype(f32)
    ne = node_emb + jnp.dot(oh_bm, both, preferred_element_type=f32)
    z1_ref[...] = jnp.dot(ne, W1_ref[...], preferred_element_type=f32)


def _tc_scale_body(deg_ref, z1_ref, y1_ref, dinv_ref):
    deg = deg_ref[0, :, :1] + deg_ref[1, :, :1] + 1.0
    dinv = lax.rsqrt(jnp.maximum(deg, 1.0))
    y1_ref[...] = dinv * z1_ref[...]
    dinv_ref[...] = dinv


def _tc_mid_body(agg_ref, y1_ref, dinv_ref, b1_ref, W2_ref, y2_ref):
    f32 = jnp.float32
    dinv = dinv_ref[...]
    h1 = jnp.maximum(dinv * (agg_ref[0] + agg_ref[1] + y1_ref[...]) + b1_ref[...], 0.0)
    z2 = jnp.dot(h1, W2_ref[...], preferred_element_type=f32)
    y2_ref[...] = dinv * z2


def _tc_out_body(agg_ref, y2_ref, dinv_ref, b2_ref, Wo_ref, bo_ref, We1_ref,
                 We2_ref, be_ref, nnp_ref, a_ref, c_ref):
    f32 = jnp.float32
    dinv = dinv_ref[...]
    h = jnp.maximum(dinv * (agg_ref[0] + agg_ref[1] + y2_ref[...]) + b2_ref[...], 0.0)
    nnp_ref[...] = jnp.dot(h, Wo_ref[...], preferred_element_type=f32) + bo_ref[...]
    a_ref[...] = jnp.dot(h, We1_ref[...], preferred_element_type=f32) + be_ref[...]
    c_ref[...] = jnp.dot(h, We2_ref[...], preferred_element_type=f32)


def kernel(x, edge_index, timestep, batch_map, obj_x, obj_pos, nucleotide_mask,
           central_mask, backbone_mask, obj_batch, W_node, b_node, W_cond,
           b_cond, W_time, b_time, W_c1, b_c1, W_c2, b_c2, W_out, b_out,
           W_edge, b_edge):
    f32 = jnp.float32
    i32 = jnp.int32
    pad5 = jnp.zeros((N, NDP - ND), f32)
    x_pad = jnp.concatenate([x, pad5], axis=1)
    cond_pad = jnp.concatenate([obj_x, obj_pos, pad5], axis=1)
    Wn_pad = jnp.concatenate([W_node, jnp.zeros((NDP - ND, HID), f32)], axis=0)
    Wo_pad = jnp.concatenate([W_out, jnp.zeros((HID, NDP - ND), f32)], axis=1)
    bo_pad = jnp.concatenate([b_out, jnp.zeros((NDP - ND,), f32)]).reshape(1, NDP)

    src3 = edge_index[0].reshape(NW, NCHUNK, ECH)
    dst3 = edge_index[1].reshape(NW, NCHUNK, ECH)

    deg = _deg_kernel(dst3)

    z1 = pl.pallas_call(
        _tc_pre_body,
        out_shape=jax.ShapeDtypeStruct((N, HID), f32),
    )(x_pad, cond_pad, timestep.reshape(G, 1),
      batch_map.reshape(N, 1), obj_batch.reshape(1, N),
      nucleotide_mask.reshape(N, 1),
      central_mask.astype(i32).reshape(N, 1),
      backbone_mask.astype(i32).reshape(N, 1),
      Wn_pad, b_node.reshape(1, HID), W_cond, b_cond.reshape(1, HID),
      W_time, b_time.reshape(1, HID), W_c1)

    y1, dinv = pl.pallas_call(
        _tc_scale_body,
        out_shape=(jax.ShapeDtypeStruct((N, HID), f32),
                   jax.ShapeDtypeStruct((N, 1), f32)),
    )(deg, z1)

    agg1 = _conv_kernel(y1, src3, dst3)

    y2 = pl.pallas_call(
        _tc_mid_body,
        out_shape=jax.ShapeDtypeStruct((N, HID), f32),
    )(agg1, y1, dinv, b_c1.reshape(1, HID), W_c2)

    agg2 = _conv_kernel(y2, src3, dst3)

    nnp, a_col, c_col = pl.pallas_call(
        _tc_out_body,
        out_shape=(jax.ShapeDtypeStruct((N, NDP), f32),
                   jax.ShapeDtypeStruct((N, 1), f32),
                   jax.ShapeDtypeStruct((N, 1), f32)),
    )(agg2, y2, dinv, b_c2.reshape(1, HID), Wo_pad, bo_pad,
      W_edge[:HID], W_edge[HID:], b_edge.reshape(1, 1))

    edge_logits = _triu_kernel(a_col.reshape(N), c_col.reshape(N),
                               jnp.asarray(_TRI_R), jnp.asarray(_TRI_C))

    return (nnp[:, :ND], edge_logits)


# speedup vs baseline: 31.8600x; 1.0307x over previous
import functools

import numpy as np
import jax
import jax.numpy as jnp
from jax import lax
from jax.experimental import pallas as pl
from jax.experimental.pallas import tpu as pltpu
from jax.experimental.pallas import tpu_sc as plsc

N = 1024
E = 32768
G = 8
ND = 131
NDP = 136
HID = 128
NC = 2
NS = 16
NW = NC * NS
EPW = E // NW
ECH = 128
NCHUNK = EPW // ECH
P = N * (N - 1) // 2
PPW = P // NW
KT = PPW // 16

_mesh = plsc.VectorSubcoreMesh(core_axis_name="c", subcore_axis_name="s")

_tri_r, _tri_c = np.triu_indices(N, k=1)
_TRI_R = np.ascontiguousarray(_tri_r.astype(np.int32))
_TRI_C = np.ascontiguousarray(_tri_c.astype(np.int32))


@functools.partial(
    pl.kernel,
    out_type=jax.ShapeDtypeStruct((NC, N, HID), jnp.float32),
    mesh=_mesh,
    compiler_params=pltpu.CompilerParams(needs_layout_passes=False),
    scratch_types=[
        pltpu.VMEM((NCHUNK, ECH), jnp.int32),
        pltpu.VMEM((ECH, HID), jnp.float32),
        pltpu.VMEM((64, HID), jnp.float32),
        pltpu.VMEM_SHARED((N, HID), jnp.float32),
    ],
)
def _deg_kernel(d_hbm, out_hbm, didx_v, ones_v, buf_v, tab_sh):
    c = lax.axis_index("c")
    s = lax.axis_index("s")
    wid = c * NS + s
    one16 = jnp.full((16,), 1.0, jnp.float32)
    zero16 = jnp.zeros((16,), jnp.float32)
    for i in range(ECH):
        for j in range(HID // 16):
            ones_v[i, pl.ds(j * 16, 16)] = one16
    for i in range(64):
        for j in range(HID // 16):
            buf_v[i, pl.ds(j * 16, 16)] = zero16
    pltpu.sync_copy(d_hbm.at[wid], didx_v)
    pltpu.sync_copy(buf_v, tab_sh.at[pl.ds(s * 64, 64)])
    plsc.subcore_barrier()
    for j in range(NCHUNK):
        pltpu.sync_copy(ones_v, tab_sh.at[didx_v.at[j]], add=True)
    plsc.subcore_barrier()
    pltpu.sync_copy(tab_sh.at[pl.ds(s * 64, 64)],
                    out_hbm.at[c, pl.ds(s * 64, 64)])


@functools.partial(
    pl.kernel,
    out_type=jax.ShapeDtypeStruct((NC, N, HID), jnp.float32),
    mesh=_mesh,
    compiler_params=pltpu.CompilerParams(needs_layout_passes=False),
    scratch_types=[
        pltpu.VMEM((NCHUNK, ECH), jnp.int32),
        pltpu.VMEM((NCHUNK, ECH), jnp.int32),
        pltpu.VMEM((ECH, HID), jnp.float32),
        pltpu.VMEM((ECH, HID), jnp.float32),
        pltpu.VMEM((64, HID), jnp.float32),
        pltpu.VMEM_SHARED((N, HID), jnp.float32),
        pltpu.SemaphoreType.DMA,
        pltpu.SemaphoreType.DMA,
    ],
)
def _conv_kernel(y_hbm, s_hbm, d_hbm, out_hbm, sidx_v, didx_v, rows_a, rows_b,
                 zbuf_v, agg_sh, sem_a, sem_b):
    c = lax.axis_index("c")
    s = lax.axis_index("s")
    wid = c * NS + s
    zero16 = jnp.zeros((16,), jnp.float32)
    for i in range(64):
        for j in range(HID // 16):
            zbuf_v[i, pl.ds(j * 16, 16)] = zero16
    pltpu.sync_copy(s_hbm.at[wid], sidx_v)
    pltpu.sync_copy(d_hbm.at[wid], didx_v)
    pltpu.sync_copy(zbuf_v, agg_sh.at[pl.ds(s * 64, 64)])
    plsc.subcore_barrier()
    bufs = (rows_a, rows_b)
    sems = (sem_a, sem_b)
    copies = [pltpu.async_copy(y_hbm.at[sidx_v.at[0]], rows_a, sem_a), None]
    for j in range(NCHUNK):
        b = j % 2
        copies[b].wait()
        if j + 1 < NCHUNK:
            nb = (j + 1) % 2
            copies[nb] = pltpu.async_copy(y_hbm.at[sidx_v.at[j + 1]],
                                          bufs[nb], sems[nb])
        pltpu.sync_copy(bufs[b], agg_sh.at[didx_v.at[j]], add=True)
    plsc.subcore_barrier()
    pltpu.sync_copy(agg_sh.at[pl.ds(s * 64, 64)],
                    out_hbm.at[c, pl.ds(s * 64, 64)])


@functools.partial(
    pl.kernel,
    out_type=jax.ShapeDtypeStruct((P,), jnp.float32),
    mesh=_mesh,
    compiler_params=pltpu.CompilerParams(needs_layout_passes=False),
    scratch_types=[
        pltpu.VMEM((N,), jnp.float32),
        pltpu.VMEM((N,), jnp.float32),
        pltpu.VMEM((PPW,), jnp.int32),
        pltpu.VMEM((PPW,), jnp.int32),
        pltpu.VMEM((PPW,), jnp.float32),
        pltpu.SemaphoreType.DMA,
        pltpu.SemaphoreType.DMA,
        pltpu.SemaphoreType.DMA,
        pltpu.SemaphoreType.DMA,
        pltpu.SemaphoreType.DMA,
    ],
)
def _triu_kernel(a_hbm, c_hbm, ri_hbm, ci_hbm, out_hbm, a_v, c_v, r_v, ci_v,
                 o_v, sem_t, sem_b0, sem_b1, sem_b2, sem_o):
    c = lax.axis_index("c")
    s = lax.axis_index("s")
    wid = c * NS + s
    NBLK = 3
    BCH = KT // NBLK
    BP = BCH * 16
    bsems = (sem_b0, sem_b1, sem_b2)
    cpa = pltpu.async_copy(a_hbm, a_v, sem_t)
    cpc = pltpu.async_copy(c_hbm, c_v, sem_t)
    rcps, ccps = [], []
    for b in range(NBLK):
        rcps.append(pltpu.async_copy(ri_hbm.at[pl.ds(wid * PPW + b * BP, BP)],
                                     r_v.at[pl.ds(b * BP, BP)], bsems[b]))
        ccps.append(pltpu.async_copy(ci_hbm.at[pl.ds(wid * PPW + b * BP, BP)],
                                     ci_v.at[pl.ds(b * BP, BP)], bsems[b]))
    cpa.wait()
    cpc.wait()

    def body(k, carry):
        off = k * 16
        r16 = r_v[pl.ds(off, 16)]
        c16 = ci_v[pl.ds(off, 16)]
        av = plsc.load_gather(a_v, [r16])
        cv = plsc.load_gather(c_v, [c16])
        o_v[pl.ds(off, 16)] = av + cv
        return carry

    ocps = []
    for b in range(NBLK):
        rcps[b].wait()
        ccps[b].wait()
        lax.fori_loop(b * BCH, (b + 1) * BCH, body, 0)
        ocps.append(pltpu.async_copy(
            o_v.at[pl.ds(b * BP, BP)],
            out_hbm.at[pl.ds(wid * PPW + b * BP, BP)], sem_o))
    for cp in ocps:
        cp.wait()


def _tc_pre_body(x_ref, cond_ref, ts_ref, bm_ref, ob_ref, nuc_ref, cen_ref,
                 bb_ref, Wn_ref, bn_ref, Wc_ref, bc_ref, Wt_ref,
                 bt_ref, W1_ref, z1_ref):
    f32 = jnp.float32
    t = ts_ref[...].astype(f32)
    j64 = lax.broadcasted_iota(jnp.int32, (1, HID // 2), 1).astype(f32)
    freqs = jnp.exp((-np.log(10000.0) / (HID // 2)) * j64)
    args = t * freqs
    time_emb = jnp.concatenate([jnp.cos(args), jnp.sin(args)], axis=1)
    Wn = Wn_ref[...]
    bn = bn_ref[...]
    node_emb = jnp.dot(x_ref[...], Wn, preferred_element_type=f32) + bn
    cond_emb = jnp.dot(cond_ref[...], Wn, preferred_element_type=f32) + bn
    nuc = nuc_ref[...]
    is_left = nuc == 0
    is_right = nuc == 2
    is_base = bb_ref[...] == 0
    is_central = cen_ref[...] != 0
    is_cond = is_left | (is_central & is_base) | (is_right & is_base)
    m = is_cond.astype(f32)
    oh_obj = (ob_ref[...] == lax.broadcasted_iota(jnp.int32, (G, N), 0))
    oh_obj = oh_obj.astype(f32)
    num = jnp.dot(oh_obj, cond_emb * m, preferred_element_type=f32, precision=lax.Precision.HIGHEST)
    cnt = jnp.dot(oh_obj, m, preferred_element_type=f32, precision=lax.Precision.HIGHEST)
    pooled = jnp.where(cnt > 0, num / jnp.maximum(cnt, 1.0), 0.0)
    time_h = jnp.dot(time_emb, Wt_ref[...], preferred_element_type=f32) + bt_ref[...]
    cond_h = jnp.dot(pooled, Wc_ref[...], preferred_element_type=f32) + bc_ref[...]
    both = time_h + cond_h
    oh_bm = (bm_ref[...] == lax.broadcasted_iota(jnp.int32, (N, G), 1)).astype(f32)
    ne = node_emb + jnp.dot(oh_bm, both, preferred_element_type=f32, precision=lax.Precision.HIGHEST)
    z1_ref[...] = jnp.dot(ne, W1_ref[...], preferred_element_type=f32)


def _tc_scale_body(deg_ref, z1_ref, y1_ref, dinv_ref):
    deg = deg_ref[0, :, :1] + deg_ref[1, :, :1] + 1.0
    dinv = lax.rsqrt(jnp.maximum(deg, 1.0))
    y1_ref[...] = dinv * z1_ref[...]
    dinv_ref[...] = dinv


def _tc_mid_body(agg_ref, y1_ref, dinv_ref, b1_ref, W2_ref, y2_ref):
    f32 = jnp.float32
    dinv = dinv_ref[...]
    h1 = jnp.maximum(dinv * (agg_ref[0] + agg_ref[1] + y1_ref[...]) + b1_ref[...], 0.0)
    z2 = jnp.dot(h1, W2_ref[...], preferred_element_type=f32)
    y2_ref[...] = dinv * z2


def _tc_out_body(agg_ref, y2_ref, dinv_ref, b2_ref, Wo_ref, bo_ref, We1_ref,
                 We2_ref, be_ref, nnp_ref, a_ref, c_ref):
    f32 = jnp.float32
    dinv = dinv_ref[...]
    h = jnp.maximum(dinv * (agg_ref[0] + agg_ref[1] + y2_ref[...]) + b2_ref[...], 0.0)
    nnp_ref[...] = jnp.dot(h, Wo_ref[...], preferred_element_type=f32) + bo_ref[...]
    a_ref[...] = jnp.dot(h, We1_ref[...], preferred_element_type=f32) + be_ref[...]
    c_ref[...] = jnp.dot(h, We2_ref[...], preferred_element_type=f32)


def kernel(x, edge_index, timestep, batch_map, obj_x, obj_pos, nucleotide_mask,
           central_mask, backbone_mask, obj_batch, W_node, b_node, W_cond,
           b_cond, W_time, b_time, W_c1, b_c1, W_c2, b_c2, W_out, b_out,
           W_edge, b_edge):
    f32 = jnp.float32
    i32 = jnp.int32
    pad5 = jnp.zeros((N, NDP - ND), f32)
    x_pad = jnp.concatenate([x, pad5], axis=1)
    cond_pad = jnp.concatenate([obj_x, obj_pos, pad5], axis=1)
    Wn_pad = jnp.concatenate([W_node, jnp.zeros((NDP - ND, HID), f32)], axis=0)
    Wo_pad = jnp.concatenate([W_out, jnp.zeros((HID, NDP - ND), f32)], axis=1)
    bo_pad = jnp.concatenate([b_out, jnp.zeros((NDP - ND,), f32)]).reshape(1, NDP)

    src3 = edge_index[0].reshape(NW, NCHUNK, ECH)
    dst3 = edge_index[1].reshape(NW, NCHUNK, ECH)

    deg = _deg_kernel(dst3)

    z1 = pl.pallas_call(
        _tc_pre_body,
        out_shape=jax.ShapeDtypeStruct((N, HID), f32),
    )(x_pad, cond_pad, timestep.reshape(G, 1),
      batch_map.reshape(N, 1), obj_batch.reshape(1, N),
      nucleotide_mask.reshape(N, 1),
      central_mask.astype(i32).reshape(N, 1),
      backbone_mask.astype(i32).reshape(N, 1),
      Wn_pad, b_node.reshape(1, HID), W_cond, b_cond.reshape(1, HID),
      W_time, b_time.reshape(1, HID), W_c1)

    y1, dinv = pl.pallas_call(
        _tc_scale_body,
        out_shape=(jax.ShapeDtypeStruct((N, HID), f32),
                   jax.ShapeDtypeStruct((N, 1), f32)),
    )(deg, z1)

    agg1 = _conv_kernel(y1, src3, dst3)

    y2 = pl.pallas_call(
        _tc_mid_body,
        out_shape=jax.ShapeDtypeStruct((N, HID), f32),
    )(agg1, y1, dinv, b_c1.reshape(1, HID), W_c2)

    agg2 = _conv_kernel(y2, src3, dst3)

    nnp, a_col, c_col = pl.pallas_call(
        _tc_out_body,
        out_shape=(jax.ShapeDtypeStruct((N, NDP), f32),
                   jax.ShapeDtypeStruct((N, 1), f32),
                   jax.ShapeDtypeStruct((N, 1), f32)),
    )(agg2, y2, dinv, b_c2.reshape(1, HID), Wo_pad, bo_pad,
      W_edge[:HID], W_edge[HID:], b_edge.reshape(1, 1))

    edge_logits = _triu_kernel(a_col.reshape(N), c_col.reshape(N),
                               jnp.asarray(_TRI_R), jnp.asarray(_TRI_C))

    return (nnp[:, :ND], edge_logits)
